# baseline (speedup 1.0000x reference)
import functools

import jax
import jax.numpy as jnp
from jax import lax
from jax.experimental import pallas as pl
from jax.experimental.pallas import tpu as pltpu
from jax.experimental.pallas import tpu_sc as plsc

D = 1024
S = 4096
R = 2 * S
V = 199
VP = 256
NW = 32

SC_SKIP = 1024
SC_ROWS = S - SC_SKIP
ROWS_PER_W = SC_ROWS // NW
CH = 48
NT = ROWS_PER_W // CH

TB = 1024
NTB = (S + SC_SKIP) // TB
VC = 2 * VP


@functools.partial(
    pl.kernel,
    mesh=plsc.VectorSubcoreMesh(core_axis_name="c", subcore_axis_name="s"),
    out_type=jax.ShapeDtypeStruct((R, D), jnp.float32),
    scratch_types=[
        pltpu.VMEM((ROWS_PER_W,), jnp.int32),
        pltpu.VMEM((CH, D), jnp.float32),
        pltpu.VMEM((CH, D), jnp.float32),
        pltpu.SemaphoreType.DMA,
        pltpu.SemaphoreType.DMA,
        pltpu.SemaphoreType.DMA,
        pltpu.SemaphoreType.DMA,
    ],
)
def _sc_image_part(idx_hbm, image_hbm, out_hbm, idx_v,
                   buf_a, buf_b, sg_a, sg_b, ss_a, ss_b):
    wid = lax.axis_index("s") * 2 + lax.axis_index("c")
    base = SC_SKIP + wid * ROWS_PER_W
    pltpu.sync_copy(idx_hbm.at[pl.ds(base, ROWS_PER_W)], idx_v)

    bufs = (buf_a, buf_b)
    sg = (sg_a, sg_b)
    ss = (ss_a, ss_b)

    def start_gather(i):
        b = i % 2
        idx_slice = idx_v.at[pl.ds(i * CH, CH)]
        return pltpu.async_copy(image_hbm.at[idx_slice], bufs[b], sg[b])

    def start_store(i):
        b = i % 2
        dst = out_hbm.at[pl.ds(S + base + i * CH, CH)]
        return pltpu.async_copy(bufs[b], dst, ss[b])

    g = [None] * NT
    s = [None] * NT
    g[0] = start_gather(0)
    for i in range(NT):
        if i + 1 < NT:
            if i >= 1:
                s[i - 1].wait()
            g[i + 1] = start_gather(i + 1)
        g[i].wait()
        s[i] = start_store(i)
    s[NT - 2].wait()
    s[NT - 1].wait()


def _tc_body(idx_ref, tab_ref, _aliased_ref, out_ref):
    idx_b = idx_ref[0, 0, :]
    cols = lax.broadcasted_iota(jnp.int32, (TB, VC), 1)
    one_hot = (idx_b[:, None] == cols).astype(jnp.bfloat16)
    out_ref[...] = jnp.dot(one_hot, tab_ref[...],
                           preferred_element_type=jnp.float32)


_tc_part = pl.pallas_call(
    _tc_body,
    grid=(NTB,),
    in_specs=[
        pl.BlockSpec((1, 1, TB), lambda i: (i, 0, 0)),
        pl.BlockSpec((VC, D), lambda i: (0, 0)),
        pl.BlockSpec(memory_space=pl.ANY),
    ],
    out_specs=pl.BlockSpec((TB, D), lambda i: (i, 0)),
    out_shape=jax.ShapeDtypeStruct((R, D), jnp.float32),
    input_output_aliases={2: 0},
)


def kernel(hidden_states, text_end_position, multiway_split_position, text_table, image_table):
    del hidden_states, multiway_split_position
    idx = text_end_position.reshape(S).astype(jnp.int32)
    part = _sc_image_part(idx, image_table)
    tabs = jnp.concatenate([
        jnp.pad(text_table, ((0, VP - V), (0, 0))),
        jnp.pad(image_table[:V], ((0, VP - V), (0, 0))),
    ]).astype(jnp.bfloat16)
    idx_tc = jnp.concatenate([idx, idx[:SC_SKIP] + VP]).reshape(NTB, 1, TB)
    out = _tc_part(idx_tc, tabs, part)
    return out.reshape(1, R, D)


# speedup vs baseline: 1.0998x; 1.0998x over previous
import functools

import jax
import jax.numpy as jnp
from jax import lax
from jax.experimental import pallas as pl
from jax.experimental.pallas import tpu as pltpu
from jax.experimental.pallas import tpu_sc as plsc

D = 1024
S = 4096
R = 2 * S
V = 199
VP = 256
NW = 32

SC_SKIP = 2048
SC_ROWS = S - SC_SKIP
ROWS_PER_W = SC_ROWS // NW
CH = 16
NT = ROWS_PER_W // CH

TB = 1024
NTB = (S + SC_SKIP) // TB
VC = 2 * VP


@functools.partial(
    pl.kernel,
    mesh=plsc.VectorSubcoreMesh(core_axis_name="c", subcore_axis_name="s"),
    out_type=jax.ShapeDtypeStruct((R, D), jnp.float32),
    scratch_types=[
        pltpu.VMEM((ROWS_PER_W,), jnp.int32),
        pltpu.VMEM((CH, D), jnp.float32),
        pltpu.VMEM((CH, D), jnp.float32),
        pltpu.SemaphoreType.DMA,
        pltpu.SemaphoreType.DMA,
        pltpu.SemaphoreType.DMA,
        pltpu.SemaphoreType.DMA,
    ],
)
def _sc_image_part(idx_hbm, image_hbm, out_hbm, idx_v,
                   buf_a, buf_b, sg_a, sg_b, ss_a, ss_b):
    wid = lax.axis_index("s") * 2 + lax.axis_index("c")
    base = SC_SKIP + wid * ROWS_PER_W
    pltpu.sync_copy(idx_hbm.at[pl.ds(base, ROWS_PER_W)], idx_v)

    bufs = (buf_a, buf_b)
    sg = (sg_a, sg_b)
    ss = (ss_a, ss_b)

    def start_gather(i):
        b = i % 2
        idx_slice = idx_v.at[pl.ds(i * CH, CH)]
        return pltpu.async_copy(image_hbm.at[idx_slice], bufs[b], sg[b])

    def start_store(i):
        b = i % 2
        dst = out_hbm.at[pl.ds(S + base + i * CH, CH)]
        return pltpu.async_copy(bufs[b], dst, ss[b])

    g = [None] * NT
    s = [None] * NT
    g[0] = start_gather(0)
    for i in range(NT):
        if i + 1 < NT:
            if i >= 1:
                s[i - 1].wait()
            g[i + 1] = start_gather(i + 1)
        g[i].wait()
        s[i] = start_store(i)
    s[NT - 2].wait()
    s[NT - 1].wait()


def _tc_body(idx_ref, tab_ref, _aliased_ref, out_ref):
    idx_b = idx_ref[0, 0, :]
    cols = lax.broadcasted_iota(jnp.int32, (TB, VC), 1)
    one_hot = (idx_b[:, None] == cols).astype(jnp.bfloat16)
    out_ref[...] = jnp.dot(one_hot, tab_ref[...],
                           preferred_element_type=jnp.float32)


_tc_part = pl.pallas_call(
    _tc_body,
    grid=(NTB,),
    in_specs=[
        pl.BlockSpec((1, 1, TB), lambda i: (i, 0, 0)),
        pl.BlockSpec((VC, D), lambda i: (0, 0)),
        pl.BlockSpec(memory_space=pl.ANY),
    ],
    out_specs=pl.BlockSpec((TB, D), lambda i: (i, 0)),
    out_shape=jax.ShapeDtypeStruct((R, D), jnp.float32),
    input_output_aliases={2: 0},
)


def kernel(hidden_states, text_end_position, multiway_split_position, text_table, image_table):
    del hidden_states, multiway_split_position
    idx = text_end_position.reshape(S).astype(jnp.int32)
    part = _sc_image_part(idx, image_table)
    tabs = jnp.concatenate([
        jnp.pad(text_table, ((0, VP - V), (0, 0))),
        jnp.pad(image_table[:V], ((0, VP - V), (0, 0))),
    ]).astype(jnp.bfloat16)
    idx_tc = jnp.concatenate([idx, idx[:SC_SKIP] + VP]).reshape(NTB, 1, TB)
    out = _tc_part(idx_tc, tabs, part)
    return out.reshape(1, R, D)
